# baseline (speedup 1.0000x reference)
import jax
import jax.numpy as jnp
from jax import lax
from jax.experimental import pallas as pl
from jax.experimental.pallas import tpu as pltpu
from jax.experimental.pallas import tpu_sc as plsc

N_SEQ_LEN = 20
NUM_EMBEDDINGS = 100000
EMBEDDING_DIM = 32
BATCH = 4096

_CHUNK = 128
_NCHUNK = BATCH // _CHUNK
_GROUP = 4
_NGROUP = N_SEQ_LEN // _GROUP


def _body(idx_hbm, tab_hbm, out_hbm, idx_v, row_v, isem, gsem):
    nc = 2
    e = lax.axis_index("s") * nc + lax.axis_index("c")

    def task(d, carry):
        row = d * EMBEDDING_DIM + e
        pltpu.async_copy(idx_hbm.at[d], idx_v, isem).wait()
        copies = [
            pltpu.async_copy(
                tab_hbm.at[row].at[idx_v.at[r]], row_v.at[r], gsem
            )
            for r in range(_NCHUNK)
        ]
        for cp in copies:
            cp.wait()
        pltpu.async_copy(row_v, out_hbm.at[row], isem).wait()
        return carry

    lax.fori_loop(0, _GROUP, task, 0)


@jax.jit
def kernel(x, tables):
    idx = x.astype(jnp.int32).T.reshape(N_SEQ_LEN, _NCHUNK, _CHUNK)

    mesh = plsc.VectorSubcoreMesh(core_axis_name="c", subcore_axis_name="s")
    run = pl.kernel(
        _body,
        mesh=mesh,
        compiler_params=pltpu.CompilerParams(use_tc_tiling_on_sc=False),
        out_type=jax.ShapeDtypeStruct(
            (_GROUP * EMBEDDING_DIM, _NCHUNK, _CHUNK), jnp.float32
        ),
        scratch_types=[
            pltpu.VMEM((_NCHUNK, _CHUNK), jnp.int32),
            pltpu.VMEM((_NCHUNK, _CHUNK), jnp.float32),
            pltpu.SemaphoreType.DMA,
            pltpu.SemaphoreType.DMA,
        ],
    )

    outs = []
    for g in range(_NGROUP):
        d0 = g * _GROUP
        tab_g = jnp.transpose(
            tables[d0:d0 + _GROUP], (0, 2, 1)
        ).reshape(_GROUP * EMBEDDING_DIM, NUM_EMBEDDINGS)
        out_g = run(idx[d0:d0 + _GROUP], tab_g)
        outs.append(out_g.reshape(_GROUP, EMBEDDING_DIM, BATCH))

    out = jnp.concatenate(outs, axis=0)
    return jnp.transpose(out, (2, 0, 1))


# speedup vs baseline: 1.1231x; 1.1231x over previous
import jax
import jax.numpy as jnp
from jax import lax
from jax.experimental import pallas as pl
from jax.experimental.pallas import tpu as pltpu
from jax.experimental.pallas import tpu_sc as plsc

N_SEQ_LEN = 20
NUM_EMBEDDINGS = 100000
EMBEDDING_DIM = 32
BATCH = 4096

_CHUNK = 128
_NCHUNK = BATCH // _CHUNK
_ROWS = N_SEQ_LEN * EMBEDDING_DIM
_STRIPE = 8
_HALF = _ROWS // _STRIPE // 2


def _body(idx_hbm, tab_hbm, out_hbm, shared, idx_v, row_v,
          ssem, psem, gsem, osem):
    cid = lax.axis_index("c")
    sid = lax.axis_index("s")
    s = sid // 2
    rh = sid % 2

    def stage(i):
        rg = i * 2 + cid
        pltpu.async_copy(
            tab_hbm.at[pl.ds(rg * _STRIPE, _STRIPE)], shared.at[i % 2], ssem
        )

    def idxload(i):
        rg = i * 2 + cid
        d = rg // (EMBEDDING_DIM // _STRIPE)
        pltpu.async_copy(
            idx_hbm.at[d, pl.ds(rh * 16, 16)], idx_v.at[i % 2], psem
        )

    @pl.when(sid == 0)
    def _stage0():
        stage(0)

    idxload(0)

    def step(i, carry):
        @pl.when((sid == 0) & (i + 1 < _HALF))
        def _prefetch_stripe():
            stage(i + 1)

        @pl.when(i + 1 < _HALF)
        def _prefetch_idx():
            idxload(i + 1)

        @pl.when(sid == 0)
        def _wait_stripe():
            pltpu.make_async_copy(
                tab_hbm.at[pl.ds(0, _STRIPE)], shared.at[0], ssem
            ).wait()

        pltpu.make_async_copy(
            idx_hbm.at[0, pl.ds(0, 16)], idx_v.at[0], psem
        ).wait()

        plsc.subcore_barrier()

        rg = i * 2 + cid
        row = rg * _STRIPE + s
        src = shared.at[i % 2, s]
        copies = [
            pltpu.async_copy(
                src.at[idx_v.at[i % 2, r]], row_v.at[r], gsem
            )
            for r in range(16)
        ]
        for cp in copies:
            cp.wait()
        pltpu.async_copy(
            row_v, out_hbm.at[row, pl.ds(rh * 16, 16)], osem
        ).wait()

        plsc.subcore_barrier()
        return carry

    lax.fori_loop(0, _HALF, step, 0)


@jax.jit
def kernel(x, tables):
    idx = x.astype(jnp.int32).T.reshape(N_SEQ_LEN, _NCHUNK, _CHUNK)
    tab = jnp.transpose(tables, (0, 2, 1)).reshape(_ROWS, NUM_EMBEDDINGS)

    mesh = plsc.VectorSubcoreMesh(core_axis_name="c", subcore_axis_name="s")
    run = pl.kernel(
        _body,
        mesh=mesh,
        compiler_params=pltpu.CompilerParams(use_tc_tiling_on_sc=False),
        out_type=jax.ShapeDtypeStruct(
            (_ROWS, _NCHUNK, _CHUNK), jnp.float32
        ),
        scratch_types=[
            pltpu.VMEM_SHARED((2, _STRIPE, NUM_EMBEDDINGS), jnp.float32),
            pltpu.VMEM((2, 16, _CHUNK), jnp.int32),
            pltpu.VMEM((16, _CHUNK), jnp.float32),
            pltpu.SemaphoreType.DMA,
            pltpu.SemaphoreType.DMA,
            pltpu.SemaphoreType.DMA,
            pltpu.SemaphoreType.DMA,
        ],
    )
    out = run(idx, tab)
    out = out.reshape(N_SEQ_LEN, EMBEDDING_DIM, BATCH)
    return jnp.transpose(out, (2, 0, 1))


# speedup vs baseline: 1.1612x; 1.0339x over previous
import jax
import jax.numpy as jnp
from jax import lax
from jax.experimental import pallas as pl
from jax.experimental.pallas import tpu as pltpu
from jax.experimental.pallas import tpu_sc as plsc

N_SEQ_LEN = 20
NUM_EMBEDDINGS = 100000
EMBEDDING_DIM = 32
BATCH = 4096

_CHUNK = 128
_NCHUNK = BATCH // _CHUNK
_ROWS = N_SEQ_LEN * EMBEDDING_DIM


def _body(idx_hbm, tab_hbm, out_hbm, idx_v, row_v,
          psem, gsem0, gsem1, osem0, osem1):
    nc = 2
    e = lax.axis_index("s") * nc + lax.axis_index("c")

    def idxload(d):
        pltpu.async_copy(idx_hbm.at[d], idx_v.at[d % 2], psem)

    def fire_gathers(d, gsem):
        row = d * EMBEDDING_DIM + e
        for r in range(_NCHUNK):
            pltpu.async_copy(
                tab_hbm.at[row].at[idx_v.at[d % 2, r]],
                row_v.at[d % 2, r],
                gsem,
            )

    def drain_gathers(gsem):
        for _ in range(_NCHUNK):
            pltpu.make_async_copy(
                out_hbm.at[0, 0], row_v.at[0, 0], gsem
            ).wait()

    def fire_store(d, osem):
        row = d * EMBEDDING_DIM + e
        pltpu.async_copy(row_v.at[d % 2], out_hbm.at[row], osem)

    def drain_store(osem):
        pltpu.make_async_copy(out_hbm.at[0], row_v.at[0], osem).wait()

    idxload(0)

    def step(d, carry):
        @pl.when(d >= 1)
        def _retire_prev():
            @pl.when(d % 2 == 1)
            def _():
                drain_gathers(gsem0)
                fire_store(d - 1, osem0)

            @pl.when(d % 2 == 0)
            def _():
                drain_gathers(gsem1)
                fire_store(d - 1, osem1)

        pltpu.make_async_copy(idx_hbm.at[0], idx_v.at[0], psem).wait()

        @pl.when(d + 1 < N_SEQ_LEN)
        def _prefetch():
            idxload(d + 1)

        @pl.when(d >= 2)
        def _free_buf():
            @pl.when(d % 2 == 0)
            def _():
                drain_store(osem0)

            @pl.when(d % 2 == 1)
            def _():
                drain_store(osem1)

        @pl.when(d % 2 == 0)
        def _fire_even():
            fire_gathers(d, gsem0)

        @pl.when(d % 2 == 1)
        def _fire_odd():
            fire_gathers(d, gsem1)

        return carry

    lax.fori_loop(0, N_SEQ_LEN, step, 0)

    drain_gathers(gsem1)
    fire_store(N_SEQ_LEN - 1, osem1)
    drain_store(osem0)
    drain_store(osem1)


@jax.jit
def kernel(x, tables):
    idx = x.astype(jnp.int32).T.reshape(N_SEQ_LEN, _NCHUNK, _CHUNK)
    tab = jnp.transpose(tables, (0, 2, 1)).reshape(_ROWS, NUM_EMBEDDINGS)

    mesh = plsc.VectorSubcoreMesh(core_axis_name="c", subcore_axis_name="s")
    run = pl.kernel(
        _body,
        mesh=mesh,
        compiler_params=pltpu.CompilerParams(use_tc_tiling_on_sc=False),
        out_type=jax.ShapeDtypeStruct(
            (_ROWS, _NCHUNK, _CHUNK), jnp.float32
        ),
        scratch_types=[
            pltpu.VMEM((2, _NCHUNK, _CHUNK), jnp.int32),
            pltpu.VMEM((2, _NCHUNK, _CHUNK), jnp.float32),
            pltpu.SemaphoreType.DMA,
            pltpu.SemaphoreType.DMA,
            pltpu.SemaphoreType.DMA,
            pltpu.SemaphoreType.DMA,
            pltpu.SemaphoreType.DMA,
        ],
    )
    out = run(idx, tab)
    out = out.reshape(N_SEQ_LEN, EMBEDDING_DIM, BATCH)
    return jnp.transpose(out, (2, 0, 1))
